# baseline (speedup 1.0000x reference)
import functools

import jax
import jax.numpy as jnp
from jax import lax
from jax.experimental import pallas as pl
from jax.experimental.pallas import tpu as pltpu
from jax.experimental.pallas import tpu_sc as plsc

N_NODES = 10000
N_EDGES = 320000
D = 128

NC = 2
NS = 16
NW = NC * NS

EPT = N_EDGES // NW
CH = 128
EPT_P = 10240
NCHUNK = EPT_P // CH
NIB = NCHUNK // 8
TRASH = N_NODES
SPA = 10112
RPT = SPA // NS
UNROLL = 4


def _sc_aggregate(x, src, dst, zrows):
  mesh = plsc.VectorSubcoreMesh(core_axis_name="c", subcore_axis_name="s")

  @functools.partial(
      pl.kernel,
      out_type=jax.ShapeDtypeStruct((NC, SPA, D), jnp.float32),
      mesh=mesh,
      compiler_params=pltpu.CompilerParams(use_tc_tiling_on_sc=False),
      scratch_types=[
          pltpu.VMEM((2, 8, CH), jnp.int32),
          pltpu.VMEM((2, 8, CH), jnp.int32),
          pltpu.VMEM((2, CH, D), jnp.float32),
          pltpu.VMEM_SHARED((SPA, D), jnp.float32),
          pltpu.SemaphoreType.DMA,
          pltpu.SemaphoreType.DMA,
          pltpu.SemaphoreType.DMA,
          pltpu.SemaphoreType.DMA,
          pltpu.SemaphoreType.DMA,
      ],
  )
  def agg_kernel(x_hbm, src_hbm, dst_hbm, z_hbm, out_hbm,
                 sring, dring, rows_v, acc_sh,
                 isem0, isem1, gsem0, gsem1, ssem):
    c = lax.axis_index("c")
    s = lax.axis_index("s")
    w = s * NC + c
    isems = (isem0, isem1)
    gsems = (gsem0, gsem1)

    pltpu.sync_copy(z_hbm.at[pl.ds(s * RPT, RPT)],
                    acc_sh.at[pl.ds(s * RPT, RPT)])
    plsc.subcore_barrier()

    def load_idx(b, bs, wait):
      a = pltpu.make_async_copy(src_hbm.at[w, b], sring.at[bs], isems[bs])
      d = pltpu.make_async_copy(dst_hbm.at[w, b], dring.at[bs], isems[bs])
      if wait:
        a.wait()
        d.wait()
      else:
        a.start()
        d.start()

    def issue_rows(bs, k, buf):
      def issue(j2, _):
        iv = sring[bs, k, pl.ds(j2 * 16, 16)]
        for u in range(16):
          pltpu.async_copy(x_hbm.at[pl.ds(iv[u], 1)],
                           rows_v.at[buf, pl.ds(j2 * 16 + u, 1)],
                           gsems[buf])
        return ()

      lax.fori_loop(0, CH // 16, issue, ())

    def do_chunk(b, bs, k, last, next_load):
      buf = k % 2
      if not last:
        if k < 7:
          issue_rows(bs, k + 1, 1 - buf)
        else:
          load_idx(b + 1, 1 - bs, True)
          issue_rows(1 - bs, 0, 1 - buf)
      pltpu.make_async_copy(x_hbm.at[pl.ds(0, CH)], rows_v.at[buf],
                            gsems[buf]).wait()
      pltpu.async_copy(rows_v.at[buf], acc_sh.at[dring.at[bs, k]],
                       ssem, add=True).wait()
      if k == 7 and next_load:
        load_idx(b + 2, bs, False)

    load_idx(0, 0, False)
    load_idx(0, 0, True)
    load_idx(1, 1, False)
    issue_rows(0, 0, 0)

    def block_pair(bp, _):
      for half in (0, 1):
        b = bp * 2 + half
        for k in range(8):
          do_chunk(b, half, k, False, True)
      return ()

    lax.fori_loop(0, (NIB - 2) // 2, block_pair, ())
    for b_tail, bs_tail in ((NIB - 2, 0), (NIB - 1, 1)):
      for k in range(8):
        do_chunk(b_tail, bs_tail, k,
                 b_tail == NIB - 1 and k == 7, False)

    plsc.subcore_barrier()
    pltpu.sync_copy(acc_sh.at[pl.ds(s * RPT, RPT)],
                    out_hbm.at[c, pl.ds(s * RPT, RPT)])

  return agg_kernel(x, src, dst, zrows)


ROW_BLK = 1000


def _mlp_kernel(x_ref, a_ref, w1_ref, b1_ref, w2_ref, b2_ref, out_ref):
  h = x_ref[...] + a_ref[0] + a_ref[1]
  h = lax.dot_general(h, w1_ref[...], (((1,), (1,)), ((), ())),
                      preferred_element_type=jnp.float32) + b1_ref[...]
  h = jnp.maximum(h, 0.0)
  out_ref[...] = lax.dot_general(h, w2_ref[...], (((1,), (1,)), ((), ())),
                                 preferred_element_type=jnp.float32) + b2_ref[...]


def _tc_mlp(x, agg, W1, b1, W2, b2):
  grid = (N_NODES // ROW_BLK,)
  blk = lambda i: (i, 0)
  fixed = lambda i: (0, 0)
  return pl.pallas_call(
      _mlp_kernel,
      grid=grid,
      in_specs=[
          pl.BlockSpec((ROW_BLK, D), blk),
          pl.BlockSpec((NC, ROW_BLK, D), lambda i: (0, i, 0)),
          pl.BlockSpec((D, D), fixed),
          pl.BlockSpec((1, D), fixed),
          pl.BlockSpec((D, D), fixed),
          pl.BlockSpec((1, D), fixed),
      ],
      out_specs=pl.BlockSpec((ROW_BLK, D), blk),
      out_shape=jax.ShapeDtypeStruct((N_NODES, D), jnp.float32),
  )(x, agg, W1, b1, W2, b2)


@jax.jit
def kernel(x, edge_index, W1, b1, W2, b2):
  pad = EPT_P - EPT
  src = edge_index[0].astype(jnp.int32).reshape(NW, EPT)
  dst = edge_index[1].astype(jnp.int32).reshape(NW, EPT)
  src = jnp.pad(src, ((0, 0), (0, pad))).reshape(NW, NIB, 8, CH)
  dst = jnp.pad(dst, ((0, 0), (0, pad)),
                constant_values=TRASH).reshape(NW, NIB, 8, CH)
  zrows = jnp.zeros((SPA, D), jnp.float32)
  agg = _sc_aggregate(x, src, dst, zrows)
  return _tc_mlp(x, agg, W1, b1.reshape(1, D), W2, b2.reshape(1, D))


# speedup vs baseline: 1.0099x; 1.0099x over previous
import functools

import jax
import jax.numpy as jnp
from jax import lax
from jax.experimental import pallas as pl
from jax.experimental.pallas import tpu as pltpu
from jax.experimental.pallas import tpu_sc as plsc

N_NODES = 10000
N_EDGES = 320000
D = 128

NC = 2
NS = 16
NW = NC * NS

EPT = N_EDGES // NW
CH = 128
EPT_P = 10240
NCHUNK = EPT_P // CH
NIB = NCHUNK // 8
TRASH = N_NODES
SPA = 10112
RPT = SPA // NS
UNROLL = 4


def _sc_aggregate(x, src, dst, zrows):
  mesh = plsc.VectorSubcoreMesh(core_axis_name="c", subcore_axis_name="s")

  @functools.partial(
      pl.kernel,
      out_type=jax.ShapeDtypeStruct((NC, SPA, D), jnp.float32),
      mesh=mesh,
      compiler_params=pltpu.CompilerParams(use_tc_tiling_on_sc=False),
      scratch_types=[
          pltpu.VMEM((2, 8, CH), jnp.int32),
          pltpu.VMEM((2, 8, CH), jnp.int32),
          pltpu.VMEM((2, CH, D), jnp.float32),
          pltpu.VMEM_SHARED((SPA, D), jnp.float32),
          pltpu.SemaphoreType.DMA,
          pltpu.SemaphoreType.DMA,
          pltpu.SemaphoreType.DMA,
          pltpu.SemaphoreType.DMA,
          pltpu.SemaphoreType.DMA,
      ],
  )
  def agg_kernel(x_hbm, src_hbm, dst_hbm, z_hbm, out_hbm,
                 sring, dring, rows_v, acc_sh,
                 isem0, isem1, gsem0, gsem1, ssem):
    c = lax.axis_index("c")
    s = lax.axis_index("s")
    w = s * NC + c
    isems = (isem0, isem1)
    gsems = (gsem0, gsem1)

    pltpu.sync_copy(z_hbm.at[pl.ds(s * RPT, RPT)],
                    acc_sh.at[pl.ds(s * RPT, RPT)])
    plsc.subcore_barrier()

    def load_idx(b, bs, wait):
      a = pltpu.make_async_copy(src_hbm.at[w, b], sring.at[bs], isems[bs])
      d = pltpu.make_async_copy(dst_hbm.at[w, b], dring.at[bs], isems[bs])
      if wait:
        a.wait()
        d.wait()
      else:
        a.start()
        d.start()

    def issue_rows(bs, k, buf):
      if buf == 0:
        pltpu.async_copy(x_hbm.at[sring.at[bs, k]], rows_v.at[buf],
                         gsems[buf])
        return

      def issue(j2, _):
        iv = sring[bs, k, pl.ds(j2 * 16, 16)]
        for u in range(16):
          pltpu.async_copy(x_hbm.at[pl.ds(iv[u], 1)],
                           rows_v.at[buf, pl.ds(j2 * 16 + u, 1)],
                           gsems[buf])
        return ()

      lax.fori_loop(0, CH // 16, issue, ())

    def drain_rows(bs, k, buf):
      if buf == 0:
        pltpu.make_async_copy(x_hbm.at[sring.at[bs, k]], rows_v.at[buf],
                              gsems[buf]).wait()
      else:
        pltpu.make_async_copy(x_hbm.at[pl.ds(0, CH)], rows_v.at[buf],
                              gsems[buf]).wait()

    def do_chunk(b, bs, k, last, next_load):
      buf = k % 2
      if not last:
        if k < 7:
          issue_rows(bs, k + 1, 1 - buf)
        else:
          load_idx(b + 1, 1 - bs, True)
          issue_rows(1 - bs, 0, 1 - buf)
      drain_rows(bs, k, buf)
      pltpu.async_copy(rows_v.at[buf], acc_sh.at[dring.at[bs, k]],
                       ssem, add=True).wait()
      if k == 7 and next_load:
        load_idx(b + 2, bs, False)

    load_idx(0, 0, False)
    load_idx(0, 0, True)
    load_idx(1, 1, False)
    issue_rows(0, 0, 0)

    def block_pair(bp, _):
      for half in (0, 1):
        b = bp * 2 + half
        for k in range(8):
          do_chunk(b, half, k, False, True)
      return ()

    lax.fori_loop(0, (NIB - 2) // 2, block_pair, ())
    for b_tail, bs_tail in ((NIB - 2, 0), (NIB - 1, 1)):
      for k in range(8):
        do_chunk(b_tail, bs_tail, k,
                 b_tail == NIB - 1 and k == 7, False)

    plsc.subcore_barrier()
    pltpu.sync_copy(acc_sh.at[pl.ds(s * RPT, RPT)],
                    out_hbm.at[c, pl.ds(s * RPT, RPT)])

  return agg_kernel(x, src, dst, zrows)


ROW_BLK = 1000


def _mlp_kernel(x_ref, a_ref, w1_ref, b1_ref, w2_ref, b2_ref, out_ref):
  h = x_ref[...] + a_ref[0] + a_ref[1]
  h = lax.dot_general(h, w1_ref[...], (((1,), (1,)), ((), ())),
                      preferred_element_type=jnp.float32) + b1_ref[...]
  h = jnp.maximum(h, 0.0)
  out_ref[...] = lax.dot_general(h, w2_ref[...], (((1,), (1,)), ((), ())),
                                 preferred_element_type=jnp.float32) + b2_ref[...]


def _tc_mlp(x, agg, W1, b1, W2, b2):
  grid = (N_NODES // ROW_BLK,)
  blk = lambda i: (i, 0)
  fixed = lambda i: (0, 0)
  return pl.pallas_call(
      _mlp_kernel,
      grid=grid,
      in_specs=[
          pl.BlockSpec((ROW_BLK, D), blk),
          pl.BlockSpec((NC, ROW_BLK, D), lambda i: (0, i, 0)),
          pl.BlockSpec((D, D), fixed),
          pl.BlockSpec((1, D), fixed),
          pl.BlockSpec((D, D), fixed),
          pl.BlockSpec((1, D), fixed),
      ],
      out_specs=pl.BlockSpec((ROW_BLK, D), blk),
      out_shape=jax.ShapeDtypeStruct((N_NODES, D), jnp.float32),
  )(x, agg, W1, b1, W2, b2)


@jax.jit
def kernel(x, edge_index, W1, b1, W2, b2):
  pad = EPT_P - EPT
  src = edge_index[0].astype(jnp.int32).reshape(NW, EPT)
  dst = edge_index[1].astype(jnp.int32).reshape(NW, EPT)
  src = jnp.pad(src, ((0, 0), (0, pad))).reshape(NW, NIB, 8, CH)
  dst = jnp.pad(dst, ((0, 0), (0, pad)),
                constant_values=TRASH).reshape(NW, NIB, 8, CH)
  zrows = jnp.zeros((SPA, D), jnp.float32)
  agg = _sc_aggregate(x, src, dst, zrows)
  return _tc_mlp(x, agg, W1, b1.reshape(1, D), W2, b2.reshape(1, D))


# speedup vs baseline: 1.0334x; 1.0233x over previous
import functools

import jax
import jax.numpy as jnp
from jax import lax
from jax.experimental import pallas as pl
from jax.experimental.pallas import tpu as pltpu
from jax.experimental.pallas import tpu_sc as plsc

N_NODES = 10000
N_EDGES = 320000
D = 128

NC = 2
NS = 16
NW = NC * NS

EPT = N_EDGES // NW
CH = 64
EPT_P = 10240
NCHUNK = EPT_P // CH
BPB = 8
NIB = NCHUNK // BPB
NBUF = 4
TRASH = N_NODES
SPA = 10112
RPT = SPA // NS


def _sc_aggregate(x, src, dst, zrows):
  mesh = plsc.VectorSubcoreMesh(core_axis_name="c", subcore_axis_name="s")

  @functools.partial(
      pl.kernel,
      out_type=jax.ShapeDtypeStruct((NC, SPA, D), jnp.float32),
      mesh=mesh,
      scratch_types=[
          pltpu.VMEM((2, BPB, CH), jnp.int32),
          pltpu.VMEM((2, BPB, CH), jnp.int32),
          pltpu.VMEM((NBUF, CH, D), jnp.float32),
          pltpu.VMEM_SHARED((SPA, D), jnp.float32),
          pltpu.SemaphoreType.DMA,
          pltpu.SemaphoreType.DMA,
          pltpu.SemaphoreType.DMA,
          pltpu.SemaphoreType.DMA,
          pltpu.SemaphoreType.DMA,
          pltpu.SemaphoreType.DMA,
          pltpu.SemaphoreType.DMA,
          pltpu.SemaphoreType.DMA,
          pltpu.SemaphoreType.DMA,
          pltpu.SemaphoreType.DMA,
      ],
  )
  def agg_kernel(x_hbm, src_hbm, dst_hbm, z_hbm, out_hbm,
                 sring, dring, rows_v, acc_sh,
                 isem0, isem1, g0, g1, g2, g3, s0, s1, s2, s3):
    c = lax.axis_index("c")
    s = lax.axis_index("s")
    w = s * NC + c
    isems = (isem0, isem1)
    gsems = (g0, g1, g2, g3)
    ssems = (s0, s1, s2, s3)

    pltpu.sync_copy(z_hbm.at[pl.ds(s * RPT, RPT)],
                    acc_sh.at[pl.ds(s * RPT, RPT)])
    plsc.subcore_barrier()

    def load_idx(b, wait):
      bs = b % 2
      a = pltpu.make_async_copy(src_hbm.at[w, b], sring.at[bs], isems[bs])
      d = pltpu.make_async_copy(dst_hbm.at[w, b], dring.at[bs], isems[bs])
      if wait:
        a.wait()
        d.wait()
      else:
        a.start()
        d.start()

    def start_gather(g):
      b, k = g // BPB, g % BPB
      pltpu.async_copy(x_hbm.at[sring.at[b % 2, k]], rows_v.at[g % NBUF],
                       gsems[g % NBUF])

    def wait_gather(g):
      b, k = g // BPB, g % BPB
      pltpu.make_async_copy(x_hbm.at[sring.at[b % 2, k]],
                            rows_v.at[g % NBUF], gsems[g % NBUF]).wait()

    def start_scatter(g):
      b, k = g // BPB, g % BPB
      pltpu.async_copy(rows_v.at[g % NBUF], acc_sh.at[dring.at[b % 2, k]],
                       ssems[g % NBUF], add=True)

    def wait_scatter(g):
      b, k = g // BPB, g % BPB
      pltpu.make_async_copy(rows_v.at[g % NBUF],
                            acc_sh.at[dring.at[b % 2, k]],
                            ssems[g % NBUF]).wait()

    load_idx(0, False)
    load_idx(0, True)
    load_idx(1, False)
    for g in range(NBUF - 1):
      start_gather(g)

    for g in range(NCHUNK):
      b, k = g // BPB, g % BPB
      if g >= 1:
        wait_scatter(g - 1)
      if k == 0 and b >= 1 and b + 1 < NIB:
        load_idx(b + 1, False)
      gn = g + NBUF - 1
      if gn < NCHUNK:
        nb, nk = gn // BPB, gn % BPB
        if nk == 0:
          load_idx(nb, True)
        start_gather(gn)
      wait_gather(g)
      start_scatter(g)

    wait_scatter(NCHUNK - 1)

    plsc.subcore_barrier()
    pltpu.sync_copy(acc_sh.at[pl.ds(s * RPT, RPT)],
                    out_hbm.at[c, pl.ds(s * RPT, RPT)])

  return agg_kernel(x, src, dst, zrows)


ROW_BLK = 1000


def _mlp_kernel(x_ref, a_ref, w1_ref, b1_ref, w2_ref, b2_ref, out_ref):
  h = x_ref[...] + a_ref[0] + a_ref[1]
  h = lax.dot_general(h, w1_ref[...], (((1,), (1,)), ((), ())),
                      preferred_element_type=jnp.float32) + b1_ref[...]
  h = jnp.maximum(h, 0.0)
  out_ref[...] = lax.dot_general(h, w2_ref[...], (((1,), (1,)), ((), ())),
                                 preferred_element_type=jnp.float32) + b2_ref[...]


def _tc_mlp(x, agg, W1, b1, W2, b2):
  grid = (N_NODES // ROW_BLK,)
  blk = lambda i: (i, 0)
  fixed = lambda i: (0, 0)
  return pl.pallas_call(
      _mlp_kernel,
      grid=grid,
      in_specs=[
          pl.BlockSpec((ROW_BLK, D), blk),
          pl.BlockSpec((NC, ROW_BLK, D), lambda i: (0, i, 0)),
          pl.BlockSpec((D, D), fixed),
          pl.BlockSpec((1, D), fixed),
          pl.BlockSpec((D, D), fixed),
          pl.BlockSpec((1, D), fixed),
      ],
      out_specs=pl.BlockSpec((ROW_BLK, D), blk),
      out_shape=jax.ShapeDtypeStruct((N_NODES, D), jnp.float32),
  )(x, agg, W1, b1, W2, b2)


@jax.jit
def kernel(x, edge_index, W1, b1, W2, b2):
  pad = EPT_P - EPT
  src = edge_index[0].astype(jnp.int32).reshape(NW, EPT)
  dst = edge_index[1].astype(jnp.int32).reshape(NW, EPT)
  src = jnp.pad(src, ((0, 0), (0, pad))).reshape(NW, NIB, BPB, CH)
  dst = jnp.pad(dst, ((0, 0), (0, pad)),
                constant_values=TRASH).reshape(NW, NIB, BPB, CH)
  zrows = jnp.zeros((SPA, D), jnp.float32)
  agg = _sc_aggregate(x, src, dst, zrows)
  return _tc_mlp(x, agg, W1, b1.reshape(1, D), W2, b2.reshape(1, D))


# speedup vs baseline: 2.1234x; 2.0548x over previous
import functools

import jax
import jax.numpy as jnp
from jax import lax
from jax.experimental import pallas as pl
from jax.experimental.pallas import tpu as pltpu
from jax.experimental.pallas import tpu_sc as plsc

N_NODES = 10000
N_EDGES = 320000
D = 128
DH = 64

NC = 2
NS = 16

EPT = N_EDGES // NS
CH = 64
EPT_P = 20480
NCHUNK = EPT_P // CH
BPB = 8
NIB = NCHUNK // BPB
NBUF = 4
TRASH = N_NODES
SPA = 10112
XR = SPA
XPT = XR // NS
APT = SPA // NS


def _sc_aggregate(xh0, xh1, src, dst, zrows):
  mesh = plsc.VectorSubcoreMesh(core_axis_name="c", subcore_axis_name="s")

  @functools.partial(
      pl.kernel,
      out_type=(jax.ShapeDtypeStruct((SPA, DH), jnp.float32),
                jax.ShapeDtypeStruct((SPA, DH), jnp.float32)),
      mesh=mesh,
      compiler_params=pltpu.CompilerParams(use_tc_tiling_on_sc=False),
      scratch_types=[
          pltpu.VMEM((2, BPB, CH), jnp.int32),
          pltpu.VMEM((2, BPB, CH), jnp.int32),
          pltpu.VMEM((NBUF, CH, DH), jnp.float32),
          pltpu.VMEM_SHARED((XR, DH), jnp.float32),
          pltpu.VMEM_SHARED((SPA, DH), jnp.float32),
      ] + [pltpu.SemaphoreType.DMA] * 11,
  )
  def agg_kernel(xh0_hbm, xh1_hbm, src_hbm, dst_hbm, z_hbm,
                 out0_hbm, out1_hbm,
                 sring, dring, rows_v, xsh, ash, *sems):
    c = lax.axis_index("c")
    s = lax.axis_index("s")
    isems = sems[0:2]
    gsems = sems[2:2 + NBUF]
    ssems = sems[2 + NBUF:2 + 2 * NBUF]
    zsem = sems[2 + 2 * NBUF]
    x2 = xsh
    a2 = ash

    @pl.when(c == 0)
    def _():
      pltpu.async_copy(xh0_hbm.at[pl.ds(s * XPT, XPT)],
                       xsh.at[pl.ds(s * XPT, XPT)], zsem)

    @pl.when(c == 1)
    def _():
      pltpu.async_copy(xh1_hbm.at[pl.ds(s * XPT, XPT)],
                       xsh.at[pl.ds(s * XPT, XPT)], zsem)

    pltpu.sync_copy(z_hbm.at[pl.ds(s * APT, APT)],
                    ash.at[pl.ds(s * APT, APT)])
    pltpu.make_async_copy(xh0_hbm.at[pl.ds(s * XPT, XPT)],
                          xsh.at[pl.ds(s * XPT, XPT)], zsem).wait()
    plsc.subcore_barrier()

    def load_idx(b, bs, wait):
      a = pltpu.make_async_copy(src_hbm.at[s, b], sring.at[bs], isems[bs])
      d = pltpu.make_async_copy(dst_hbm.at[s, b], dring.at[bs], isems[bs])
      if wait:
        a.wait()
        d.wait()
      else:
        a.start()
        d.start()

    def start_gather(bs, k, buf):
      pltpu.async_copy(x2.at[sring.at[bs, k]], rows_v.at[buf], gsems[buf])

    def wait_gather(bs, k, buf):
      pltpu.make_async_copy(x2.at[sring.at[bs, k]], rows_v.at[buf],
                            gsems[buf]).wait()

    def start_scatter(bs, k, buf):
      pltpu.async_copy(rows_v.at[buf], a2.at[dring.at[bs, k]],
                       ssems[buf], add=True)

    def wait_scatter(bs, k, buf):
      pltpu.make_async_copy(rows_v.at[buf], a2.at[dring.at[bs, k]],
                            ssems[buf]).wait()

    def do_chunk(b, bs, k, first, last_block):
      buf = k % NBUF
      g_has_next = not (last_block and k + NBUF - 1 >= BPB)
      if k >= 1:
        wait_scatter(bs, k - 1, (k - 1) % NBUF)
      elif not first:
        wait_scatter(1 - bs, BPB - 1, (BPB - 1) % NBUF)
      if g_has_next:
        nk = k + NBUF - 1
        if nk < BPB:
          start_gather(bs, nk, nk % NBUF)
        else:
          if nk == BPB:
            load_idx(b + 1, 1 - bs, True)
          start_gather(1 - bs, nk - BPB, nk % NBUF)
      wait_gather(bs, k, buf)
      start_scatter(bs, k, buf)
      if k == 3 and not last_block:
        load_idx(b + 1, 1 - bs, False)

    load_idx(0, 0, False)
    load_idx(0, 0, True)
    for g in range(NBUF - 1):
      start_gather(0, g, g % NBUF)

    for k in range(BPB):
      do_chunk(0, 0, k, True, False)

    def block_pair(bp, _):
      for half, bs in ((1, 1), (2, 0)):
        b = bp * 2 + half
        for k in range(BPB):
          do_chunk(b, bs, k, False, False)
      return ()

    lax.fori_loop(0, (NIB - 2) // 2, block_pair, ())
    for k in range(BPB):
      do_chunk(NIB - 1, 1, k, False, True)
    wait_scatter(1, BPB - 1, (BPB - 1) % NBUF)

    plsc.subcore_barrier()

    @pl.when(c == 0)
    def _():
      pltpu.sync_copy(ash.at[pl.ds(s * APT, APT)],
                      out0_hbm.at[pl.ds(s * APT, APT)])

    @pl.when(c == 1)
    def _():
      pltpu.sync_copy(ash.at[pl.ds(s * APT, APT)],
                      out1_hbm.at[pl.ds(s * APT, APT)])

  return agg_kernel(xh0, xh1, src, dst, zrows)


ROW_BLK = 1000


def _mlp_kernel(x_ref, a_ref, w1_ref, b1_ref, w2_ref, b2_ref, out_ref):
  h = x_ref[...] + a_ref[...]
  h = lax.dot_general(h, w1_ref[...], (((1,), (1,)), ((), ())),
                      preferred_element_type=jnp.float32) + b1_ref[...]
  h = jnp.maximum(h, 0.0)
  out_ref[...] = lax.dot_general(h, w2_ref[...], (((1,), (1,)), ((), ())),
                                 preferred_element_type=jnp.float32) + b2_ref[...]


def _tc_mlp(x, agg, W1, b1, W2, b2):
  grid = (N_NODES // ROW_BLK,)
  blk = lambda i: (i, 0)
  fixed = lambda i: (0, 0)
  return pl.pallas_call(
      _mlp_kernel,
      grid=grid,
      in_specs=[
          pl.BlockSpec((ROW_BLK, D), blk),
          pl.BlockSpec((ROW_BLK, D), blk),
          pl.BlockSpec((D, D), fixed),
          pl.BlockSpec((1, D), fixed),
          pl.BlockSpec((D, D), fixed),
          pl.BlockSpec((1, D), fixed),
      ],
      out_specs=pl.BlockSpec((ROW_BLK, D), blk),
      out_shape=jax.ShapeDtypeStruct((N_NODES, D), jnp.float32),
  )(x, agg, W1, b1, W2, b2)


@jax.jit
def kernel(x, edge_index, W1, b1, W2, b2):
  pad = EPT_P - EPT
  src = edge_index[0].astype(jnp.int32).reshape(NS, EPT)
  dst = edge_index[1].astype(jnp.int32).reshape(NS, EPT)
  src = jnp.pad(src, ((0, 0), (0, pad))).reshape(NS, NIB, BPB, CH)
  dst = jnp.pad(dst, ((0, 0), (0, pad)),
                constant_values=TRASH).reshape(NS, NIB, BPB, CH)
  xpad = jnp.pad(x, ((0, SPA - N_NODES), (0, 0)))
  xh0 = xpad[:, :DH]
  xh1 = xpad[:, DH:]
  zrows = jnp.zeros((SPA, DH), jnp.float32)
  a0, a1 = _sc_aggregate(xh0, xh1, src, dst, zrows)
  agg = jnp.concatenate([a0[:N_NODES], a1[:N_NODES]], axis=1)
  return _tc_mlp(x, agg, W1, b1.reshape(1, D), W2, b2.reshape(1, D))
